# Initial kernel scaffold; baseline (speedup 1.0000x reference)
import functools

import jax
import jax.numpy as jnp
from jax import lax
from jax.experimental import pallas as pl
from jax.experimental.pallas import tpu as pltpu
from jax.experimental.pallas import tpu_sc as plsc

H = 128
K = 6
FS = 30
N_LANE = 10000
N_AGENT = 2000
B = 64
NCORE = 2
NSUB = 16
NW = NCORE * NSUB
C = 80


def _make_seg_sum(n_nodes, n_edges):
    e_per = n_edges // NW
    n_chunks = e_per // C
    span = n_nodes // NSUB
    mesh = plsc.VectorSubcoreMesh(
        core_axis_name="c", subcore_axis_name="s",
        num_cores=NCORE, num_subcores=NSUB)

    @functools.partial(
        pl.kernel,
        out_type=jax.ShapeDtypeStruct((NCORE * n_nodes, H), jnp.float32),
        mesh=mesh,
        scratch_types=[
            pltpu.VMEM((C,), jnp.int32),
            pltpu.VMEM((C,), jnp.int32),
            pltpu.VMEM((C, H), jnp.float32),
            pltpu.VMEM_SHARED((n_nodes, H), jnp.float32),
            pltpu.SemaphoreType.DMA,
        ],
    )
    def seg_sum(feat, src, dst, zeros, out, src_v, dst_v, rows_v, acc, sem):
        c = lax.axis_index("c")
        s = lax.axis_index("s")
        wid = s * NCORE + c
        base_r = s * span
        pltpu.sync_copy(zeros.at[pl.ds(base_r, span)], acc.at[pl.ds(base_r, span)])
        plsc.subcore_barrier()

        def body(j, carry):
            e0 = wid * e_per + j * C
            pltpu.sync_copy(src.at[pl.ds(e0, C)], src_v)
            pltpu.sync_copy(dst.at[pl.ds(e0, C)], dst_v)
            pltpu.async_copy(feat.at[src_v], rows_v, sem).wait()
            pltpu.sync_copy(rows_v, acc.at[dst_v], add=True)
            return carry

        lax.fori_loop(0, n_chunks, body, 0)
        plsc.subcore_barrier()
        pltpu.sync_copy(acc.at[pl.ds(base_r, span)],
                        out.at[pl.ds(c * n_nodes + base_r, span)])

    return seg_sum


def _make_seg_count(n_nodes, n_edges):
    e_per = n_edges // NW
    n_chunks = e_per // C
    span = n_nodes // NSUB
    mesh = plsc.VectorSubcoreMesh(
        core_axis_name="c", subcore_axis_name="s",
        num_cores=NCORE, num_subcores=NSUB)

    @functools.partial(
        pl.kernel,
        out_type=jax.ShapeDtypeStruct((NCORE * n_nodes, 16), jnp.float32),
        mesh=mesh,
        scratch_types=[
            pltpu.VMEM((C,), jnp.int32),
            pltpu.VMEM((C, 16), jnp.float32),
            pltpu.VMEM_SHARED((n_nodes, 16), jnp.float32),
        ],
    )
    def seg_count(dst, zeros, out, dst_v, ones_v, acc):
        c = lax.axis_index("c")
        s = lax.axis_index("s")
        wid = s * NCORE + c
        base_r = s * span

        def fill(i, carry):
            ones_v[i, :] = jnp.ones((16,), jnp.float32)
            return carry

        lax.fori_loop(0, C, fill, 0)
        pltpu.sync_copy(zeros.at[pl.ds(base_r, span)], acc.at[pl.ds(base_r, span)])
        plsc.subcore_barrier()

        def body(j, carry):
            e0 = wid * e_per + j * C
            pltpu.sync_copy(dst.at[pl.ds(e0, C)], dst_v)
            pltpu.sync_copy(ones_v, acc.at[dst_v], add=True)
            return carry

        lax.fori_loop(0, n_chunks, body, 0)
        plsc.subcore_barrier()
        pltpu.sync_copy(acc.at[pl.ds(base_r, span)],
                        out.at[pl.ds(c * n_nodes + base_r, span)])

    return seg_count


def _ln_relu(h, g, b):
    m = jnp.mean(h, axis=-1, keepdims=True)
    v = jnp.mean((h - m) * (h - m), axis=-1, keepdims=True)
    return jnp.maximum((h - m) / jnp.sqrt(v + 1e-5) * g + b, 0.0)


def _dot(a, b):
    return jnp.dot(a, b, preferred_element_type=jnp.float32)


def _make_subgraph(n_poly, n_pts, f_in, rows):
    grid = (n_poly // rows,)

    def body(x_ref, w1, b1, g1, d1, w2, b2, g2, d2, w3, b3, g3, d3, o_ref):
        x = x_ref[...]
        last = x[:, n_pts - 1:n_pts, :]
        if f_in == 2:
            x = x - last
        else:
            ch = lax.broadcasted_iota(jnp.int32, (1, 1, f_in), 2)
            x = x - jnp.where(ch < 2, last, 0.0)
        x = x.reshape(rows * n_pts, f_in)
        h = _ln_relu(_dot(x, w1[...]) + b1[...], g1[...], d1[...])
        h = _ln_relu(_dot(h, w2[...]) + b2[...], g2[...], d2[...])
        h = jnp.max(h.reshape(rows, n_pts, H), axis=1)
        o_ref[...] = _ln_relu(_dot(h, w3[...]) + b3[...], g3[...], d3[...])

    full2 = lambda shape: pl.BlockSpec(shape, lambda i: (0, 0))
    return pl.pallas_call(
        body,
        grid=grid,
        in_specs=[
            pl.BlockSpec((rows, n_pts, f_in), lambda i: (i, 0, 0)),
            full2((f_in, H)), full2((1, H)), full2((1, H)), full2((1, H)),
            full2((H, H)), full2((1, H)), full2((1, H)), full2((1, H)),
            full2((H, H)), full2((1, H)), full2((1, H)), full2((1, H)),
        ],
        out_specs=pl.BlockSpec((rows, H), lambda i: (i, 0)),
        out_shape=jax.ShapeDtypeStruct((n_poly, H), jnp.float32),
    )


def _make_gnn_dense(n_nodes, rows):
    grid = (n_nodes // rows,)
    nb = n_nodes // rows

    def body(node_ref, s0, s1, c0, c1, w1n, w1a, b1, g1, d1, w2, b2, g2, d2,
             o_ref):
        node = node_ref[...]
        cnt = c0[...][:, :1] + c1[...][:, :1]
        aggr = (s0[...] + s1[...]) / jnp.maximum(cnt, 1.0)
        h = _dot(node, w1n[...]) + _dot(aggr, w1a[...]) + b1[...]
        h = _ln_relu(h, g1[...], d1[...])
        h = _ln_relu(_dot(h, w2[...]) + b2[...], g2[...], d2[...])
        o_ref[...] = node + h

    full2 = lambda shape: pl.BlockSpec(shape, lambda i: (0, 0))
    return pl.pallas_call(
        body,
        grid=grid,
        in_specs=[
            pl.BlockSpec((rows, H), lambda i: (i, 0)),
            pl.BlockSpec((rows, H), lambda i: (i, 0)),
            pl.BlockSpec((rows, H), lambda i: (nb + i, 0)),
            pl.BlockSpec((rows, 16), lambda i: (i, 0)),
            pl.BlockSpec((rows, 16), lambda i: (nb + i, 0)),
            full2((H, H)), full2((H, H)), full2((1, H)), full2((1, H)),
            full2((1, H)),
            full2((H, H)), full2((1, H)), full2((1, H)), full2((1, H)),
        ],
        out_specs=pl.BlockSpec((rows, H), lambda i: (i, 0)),
        out_shape=jax.ShapeDtypeStruct((n_nodes, H), jnp.float32),
    )


def _gather_rows(feat, idx):
    def body(idx_ref, feat_ref, o_ref):
        o_ref[...] = feat_ref[...]

    grid_spec = pltpu.PrefetchScalarGridSpec(
        num_scalar_prefetch=1,
        grid=(B,),
        in_specs=[pl.BlockSpec((1, H), lambda i, idx_ref: (idx_ref[i], 0))],
        out_specs=pl.BlockSpec((1, H), lambda i, idx_ref: (i, 0)),
    )
    return pl.pallas_call(
        body,
        grid_spec=grid_spec,
        out_shape=jax.ShapeDtypeStruct((B, H), jnp.float32),
    )(idx, feat)


def _heads(tf, tlp_tiled, wt1, bt1, wt2, bt2, wp1, bp1, wp2, bp2):
    TD = K * FS * 2

    def body(tf_ref, tlp_ref, wt1r, bt1r, wt2r, bt2r, wp1r, bp1r, wp2r, bp2r,
             traj_ref, log_ref):
        t = tf_ref[...]
        ht = jnp.maximum(_dot(t, wt1r[...]) + bt1r[...], 0.0)
        traj_ref[...] = _dot(ht, wt2r[...]) + bt2r[...] + tlp_ref[...]
        hp = jnp.maximum(_dot(t, wp1r[...]) + bp1r[...], 0.0)
        log_ref[...] = _dot(hp, wp2r[...]) + bp2r[...]

    return pl.pallas_call(
        body,
        out_shape=(jax.ShapeDtypeStruct((B, TD), jnp.float32),
                   jax.ShapeDtypeStruct((B, K), jnp.float32)),
    )(tf, tlp_tiled, wt1, bt1, wt2, bt2, wp1, bp1, wp2, bp2)


def _enc_args(p):
    out = []
    for l, n in (("l1", "n1"), ("l2", "n2"), ("l3", "n3")):
        out += [p[l]["W"].T, p[l]["b"][None, :], p[n]["g"][None, :],
                p[n]["b"][None, :]]
    return out


def _gnn_args(p):
    w1t = p["l1"]["W"].T
    return [w1t[:H], w1t[H:], p["l1"]["b"][None, :], p["n1"]["g"][None, :],
            p["n1"]["b"][None, :], p["l2"]["W"].T, p["l2"]["b"][None, :],
            p["n2"]["g"][None, :], p["n2"]["b"][None, :]]


def kernel(lane_points, agent_history, edge_index_lane_to_lane,
           edge_index_agent_to_agent, edge_index_lane_to_agent,
           target_agent_global_idx, target_last_pos, params):
    e_ll, e_aa, e_la = (edge_index_lane_to_lane, edge_index_agent_to_agent,
                        edge_index_lane_to_agent)

    sub_lane = _make_subgraph(N_LANE, 20, 2, 200)
    sub_agent = _make_subgraph(N_AGENT, 20, 7, 200)
    lane_feat = sub_lane(lane_points, *_enc_args(params["lane_enc"]))
    agent_feat = sub_agent(agent_history, *_enc_args(params["agent_enc"]))

    zl = jnp.zeros((N_LANE, H), jnp.float32)
    za = jnp.zeros((N_AGENT, H), jnp.float32)
    zl16 = jnp.zeros((N_LANE, 16), jnp.float32)
    za16 = jnp.zeros((N_AGENT, 16), jnp.float32)

    cnt_ll = _make_seg_count(N_LANE, e_ll.shape[1])(e_ll[1], zl16)
    cnt_aa = _make_seg_count(N_AGENT, e_aa.shape[1])(e_aa[1], za16)
    cnt_la = _make_seg_count(N_AGENT, e_la.shape[1])(e_la[1], za16)

    sum_ll = _make_seg_sum(N_LANE, e_ll.shape[1])
    sum_aa = _make_seg_sum(N_AGENT, e_aa.shape[1])
    sum_la = _make_seg_sum(N_AGENT, e_la.shape[1])
    dense_l = _make_gnn_dense(N_LANE, 400)
    dense_a = _make_gnn_dense(N_AGENT, 400)

    for lp in params["layers"]:
        s = sum_ll(lane_feat, e_ll[0], e_ll[1], zl)
        lane_feat = dense_l(lane_feat, s, s, cnt_ll, cnt_ll,
                            *_gnn_args(lp["ll"]))
        s = sum_aa(agent_feat, e_aa[0], e_aa[1], za)
        agent_feat = dense_a(agent_feat, s, s, cnt_aa, cnt_aa,
                             *_gnn_args(lp["aa"]))
        s = sum_la(lane_feat, e_la[0], e_la[1], za)
        agent_feat = dense_a(agent_feat, s, s, cnt_la, cnt_la,
                             *_gnn_args(lp["la"]))

    tf = _gather_rows(agent_feat, target_agent_global_idx)
    tlp_tiled = jnp.tile(target_last_pos, (1, K * FS))
    traj_flat, logits = _heads(
        tf, tlp_tiled,
        params["traj1"]["W"].T, params["traj1"]["b"][None, :],
        params["traj2"]["W"].T, params["traj2"]["b"][None, :],
        params["prob1"]["W"].T, params["prob1"]["b"][None, :],
        params["prob2"]["W"].T, params["prob2"]["b"][None, :])
    pred = traj_flat.reshape(B, K, FS, 2)
    return pred, logits


# speedup vs baseline: 4.3636x; 4.3636x over previous
import functools

import jax
import jax.numpy as jnp
from jax import lax
from jax.experimental import pallas as pl
from jax.experimental.pallas import tpu as pltpu
from jax.experimental.pallas import tpu_sc as plsc

H = 128
K = 6
FS = 30
N_LANE = 10000
N_AGENT = 2000
B = 64
NCORE = 2
NSUB = 16
NW = NCORE * NSUB
C = 80


def _pad_nodes(n):
    return ((n + NSUB * 8 - 1) // (NSUB * 8)) * (NSUB * 8)


def _make_seg_sum(n_nodes, n_edges):
    e_per = n_edges // NW
    n_chunks = e_per // C
    n_pad = _pad_nodes(n_nodes)
    span = n_pad // NSUB
    mesh = plsc.VectorSubcoreMesh(
        core_axis_name="c", subcore_axis_name="s",
        num_cores=NCORE, num_subcores=NSUB)

    @functools.partial(
        pl.kernel,
        out_type=jax.ShapeDtypeStruct((NCORE, n_pad, H), jnp.float32),
        mesh=mesh,
        scratch_types=[
            pltpu.VMEM((C,), jnp.int32),
            pltpu.VMEM((C,), jnp.int32),
            pltpu.VMEM((C, H), jnp.float32),
            pltpu.VMEM_SHARED((n_pad, H), jnp.float32),
            pltpu.SemaphoreType.DMA,
        ],
    )
    def seg_sum(feat, src, dst, zeros, out, src_v, dst_v, rows_v, acc, sem):
        c = lax.axis_index("c")
        s = lax.axis_index("s")
        wid = s * NCORE + c
        base_r = s * span
        pltpu.sync_copy(zeros.at[pl.ds(base_r, span)], acc.at[pl.ds(base_r, span)])
        plsc.subcore_barrier()

        def body(j, carry):
            e0 = wid * e_per + j * C
            pltpu.sync_copy(src.at[pl.ds(e0, C)], src_v)
            pltpu.sync_copy(dst.at[pl.ds(e0, C)], dst_v)
            pltpu.async_copy(feat.at[src_v], rows_v, sem).wait()
            pltpu.sync_copy(rows_v, acc.at[dst_v], add=True)
            return carry

        lax.fori_loop(0, n_chunks, body, 0)
        plsc.subcore_barrier()
        pltpu.sync_copy(acc.at[pl.ds(base_r, span)],
                        out.at[c, pl.ds(base_r, span)])

    return seg_sum


def _make_seg_count(n_nodes, n_edges):
    e_per = n_edges // NW
    n_chunks = e_per // C
    n_pad = _pad_nodes(n_nodes)
    span = n_pad // NSUB
    mesh = plsc.VectorSubcoreMesh(
        core_axis_name="c", subcore_axis_name="s",
        num_cores=NCORE, num_subcores=NSUB)

    @functools.partial(
        pl.kernel,
        out_type=jax.ShapeDtypeStruct((NCORE, n_pad, H), jnp.float32),
        mesh=mesh,
        scratch_types=[
            pltpu.VMEM((C,), jnp.int32),
            pltpu.VMEM((C, H), jnp.float32),
            pltpu.VMEM_SHARED((n_pad, H), jnp.float32),
        ],
    )
    def seg_count(dst, zeros, ones, out, dst_v, ones_v, acc):
        c = lax.axis_index("c")
        s = lax.axis_index("s")
        wid = s * NCORE + c
        base_r = s * span
        pltpu.sync_copy(ones, ones_v)
        pltpu.sync_copy(zeros.at[pl.ds(base_r, span)], acc.at[pl.ds(base_r, span)])
        plsc.subcore_barrier()

        def body(j, carry):
            e0 = wid * e_per + j * C
            pltpu.sync_copy(dst.at[pl.ds(e0, C)], dst_v)
            pltpu.sync_copy(ones_v, acc.at[dst_v], add=True)
            return carry

        lax.fori_loop(0, n_chunks, body, 0)
        plsc.subcore_barrier()
        pltpu.sync_copy(acc.at[pl.ds(base_r, span)],
                        out.at[c, pl.ds(base_r, span)])

    return seg_count


def _ln_relu(h, g, b):
    m = jnp.mean(h, axis=-1, keepdims=True)
    v = jnp.mean((h - m) * (h - m), axis=-1, keepdims=True)
    return jnp.maximum((h - m) / jnp.sqrt(v + 1e-5) * g + b, 0.0)


def _dot(a, b):
    return jnp.dot(a, b, preferred_element_type=jnp.float32)


def _make_subgraph(n_poly, n_pts, f_in, rows):
    grid = (n_poly // rows,)

    def body(x_ref, w1, b1, g1, d1, w2, b2, g2, d2, w3, b3, g3, d3, o_ref):
        x = x_ref[...]
        last = x[:, n_pts - 1:n_pts, :]
        if f_in == 2:
            x = x - last
        else:
            ch = lax.broadcasted_iota(jnp.int32, (1, 1, f_in), 2)
            x = x - jnp.where(ch < 2, last, 0.0)
        x = x.reshape(rows * n_pts, f_in)
        h = _ln_relu(_dot(x, w1[...]) + b1[...], g1[...], d1[...])
        h = _ln_relu(_dot(h, w2[...]) + b2[...], g2[...], d2[...])
        h = jnp.max(h.reshape(rows, n_pts, H), axis=1)
        o_ref[...] = _ln_relu(_dot(h, w3[...]) + b3[...], g3[...], d3[...])

    full2 = lambda shape: pl.BlockSpec(shape, lambda i: (0, 0))
    return pl.pallas_call(
        body,
        grid=grid,
        in_specs=[
            pl.BlockSpec((rows, n_pts, f_in), lambda i: (i, 0, 0)),
            full2((f_in, H)), full2((1, H)), full2((1, H)), full2((1, H)),
            full2((H, H)), full2((1, H)), full2((1, H)), full2((1, H)),
            full2((H, H)), full2((1, H)), full2((1, H)), full2((1, H)),
        ],
        out_specs=pl.BlockSpec((rows, H), lambda i: (i, 0)),
        out_shape=jax.ShapeDtypeStruct((n_poly, H), jnp.float32),
    )


def _make_gnn_dense(n_nodes, rows):
    grid = (n_nodes // rows,)

    def body(node_ref, s0, s1, c0, c1, w1n, w1a, b1, g1, d1, w2, b2, g2, d2,
             o_ref):
        node = node_ref[...]
        cnt = c0[0][:, :1] + c1[0][:, :1]
        aggr = (s0[0] + s1[0]) / jnp.maximum(cnt, 1.0)
        h = _dot(node, w1n[...]) + _dot(aggr, w1a[...]) + b1[...]
        h = _ln_relu(h, g1[...], d1[...])
        h = _ln_relu(_dot(h, w2[...]) + b2[...], g2[...], d2[...])
        o_ref[...] = node + h

    full2 = lambda shape: pl.BlockSpec(shape, lambda i: (0, 0))
    return pl.pallas_call(
        body,
        grid=grid,
        in_specs=[
            pl.BlockSpec((rows, H), lambda i: (i, 0)),
            pl.BlockSpec((1, rows, H), lambda i: (0, i, 0)),
            pl.BlockSpec((1, rows, H), lambda i: (1, i, 0)),
            pl.BlockSpec((1, rows, H), lambda i: (0, i, 0)),
            pl.BlockSpec((1, rows, H), lambda i: (1, i, 0)),
            full2((H, H)), full2((H, H)), full2((1, H)), full2((1, H)),
            full2((1, H)),
            full2((H, H)), full2((1, H)), full2((1, H)), full2((1, H)),
        ],
        out_specs=pl.BlockSpec((rows, H), lambda i: (i, 0)),
        out_shape=jax.ShapeDtypeStruct((n_nodes, H), jnp.float32),
    )


def _gather_rows(feat, idx):
    def body(idx_ref, feat_ref, o_ref):
        o_ref[...] = feat_ref[...]

    grid_spec = pltpu.PrefetchScalarGridSpec(
        num_scalar_prefetch=1,
        grid=(B,),
        in_specs=[pl.BlockSpec((1, 1, H),
                               lambda i, idx_ref: (idx_ref[i], 0, 0))],
        out_specs=pl.BlockSpec((1, 1, H), lambda i, idx_ref: (i, 0, 0)),
    )
    out = pl.pallas_call(
        body,
        grid_spec=grid_spec,
        out_shape=jax.ShapeDtypeStruct((B, 1, H), jnp.float32),
    )(idx, feat.reshape(feat.shape[0], 1, H))
    return out.reshape(B, H)


def _heads(tf, tlp_tiled, wt1, bt1, wt2, bt2, wp1, bp1, wp2, bp2):
    TD = K * FS * 2

    def body(tf_ref, tlp_ref, wt1r, bt1r, wt2r, bt2r, wp1r, bp1r, wp2r, bp2r,
             traj_ref, log_ref):
        t = tf_ref[...]
        ht = jnp.maximum(_dot(t, wt1r[...]) + bt1r[...], 0.0)
        traj_ref[...] = _dot(ht, wt2r[...]) + bt2r[...] + tlp_ref[...]
        hp = jnp.maximum(_dot(t, wp1r[...]) + bp1r[...], 0.0)
        log_ref[...] = _dot(hp, wp2r[...]) + bp2r[...]

    return pl.pallas_call(
        body,
        out_shape=(jax.ShapeDtypeStruct((B, TD), jnp.float32),
                   jax.ShapeDtypeStruct((B, K), jnp.float32)),
    )(tf, tlp_tiled, wt1, bt1, wt2, bt2, wp1, bp1, wp2, bp2)


def _enc_args(p):
    out = []
    for l, n in (("l1", "n1"), ("l2", "n2"), ("l3", "n3")):
        out += [p[l]["W"].T, p[l]["b"][None, :], p[n]["g"][None, :],
                p[n]["b"][None, :]]
    return out


def _gnn_args(p):
    w1t = p["l1"]["W"].T
    return [w1t[:H], w1t[H:], p["l1"]["b"][None, :], p["n1"]["g"][None, :],
            p["n1"]["b"][None, :], p["l2"]["W"].T, p["l2"]["b"][None, :],
            p["n2"]["g"][None, :], p["n2"]["b"][None, :]]


def kernel(lane_points, agent_history, edge_index_lane_to_lane,
           edge_index_agent_to_agent, edge_index_lane_to_agent,
           target_agent_global_idx, target_last_pos, params):
    e_ll, e_aa, e_la = (edge_index_lane_to_lane, edge_index_agent_to_agent,
                        edge_index_lane_to_agent)

    sub_lane = _make_subgraph(N_LANE, 20, 2, 200)
    sub_agent = _make_subgraph(N_AGENT, 20, 7, 200)
    lane_feat = sub_lane(lane_points, *_enc_args(params["lane_enc"]))
    agent_feat = sub_agent(agent_history, *_enc_args(params["agent_enc"]))

    zl = jnp.zeros((_pad_nodes(N_LANE), H), jnp.float32)
    za = jnp.zeros((_pad_nodes(N_AGENT), H), jnp.float32)

    ones_c = jnp.ones((C, H), jnp.float32)
    cnt_ll = _make_seg_count(N_LANE, e_ll.shape[1])(e_ll[1], zl, ones_c)
    cnt_aa = _make_seg_count(N_AGENT, e_aa.shape[1])(e_aa[1], za, ones_c)
    cnt_la = _make_seg_count(N_AGENT, e_la.shape[1])(e_la[1], za, ones_c)

    sum_ll = _make_seg_sum(N_LANE, e_ll.shape[1])
    sum_aa = _make_seg_sum(N_AGENT, e_aa.shape[1])
    sum_la = _make_seg_sum(N_AGENT, e_la.shape[1])
    dense_l = _make_gnn_dense(N_LANE, 400)
    dense_a = _make_gnn_dense(N_AGENT, 400)

    for lp in params["layers"]:
        s = sum_ll(lane_feat, e_ll[0], e_ll[1], zl)
        lane_feat = dense_l(lane_feat, s, s, cnt_ll, cnt_ll,
                            *_gnn_args(lp["ll"]))
        s = sum_aa(agent_feat, e_aa[0], e_aa[1], za)
        agent_feat = dense_a(agent_feat, s, s, cnt_aa, cnt_aa,
                             *_gnn_args(lp["aa"]))
        s = sum_la(lane_feat, e_la[0], e_la[1], za)
        agent_feat = dense_a(agent_feat, s, s, cnt_la, cnt_la,
                             *_gnn_args(lp["la"]))

    tf = _gather_rows(agent_feat, target_agent_global_idx)
    tlp_tiled = jnp.tile(target_last_pos, (1, K * FS))
    traj_flat, logits = _heads(
        tf, tlp_tiled,
        params["traj1"]["W"].T, params["traj1"]["b"][None, :],
        params["traj2"]["W"].T, params["traj2"]["b"][None, :],
        params["prob1"]["W"].T, params["prob1"]["b"][None, :],
        params["prob2"]["W"].T, params["prob2"]["b"][None, :])
    pred = traj_flat.reshape(B, K, FS, 2)
    return pred, logits


# speedup vs baseline: 7.4981x; 1.7183x over previous
import functools

import jax
import jax.numpy as jnp
from jax import lax
from jax.experimental import pallas as pl
from jax.experimental.pallas import tpu as pltpu
from jax.experimental.pallas import tpu_sc as plsc

H = 128
K = 6
FS = 30
N_LANE = 10000
N_AGENT = 2000
B = 64
NCORE = 2
NSUB = 16
NW = NCORE * NSUB
C = 125


def _pad_nodes(n):
    return ((n + NSUB * 8 - 1) // (NSUB * 8)) * (NSUB * 8)


NB = 8


def _make_seg_sum(n_nodes, n_edges):
    e_per = n_edges // NW
    m_blocks = e_per // (NB * C)
    n_pad = _pad_nodes(n_nodes)
    span = n_pad // NSUB
    mesh = plsc.VectorSubcoreMesh(
        core_axis_name="c", subcore_axis_name="s",
        num_cores=NCORE, num_subcores=NSUB)

    @functools.partial(
        pl.kernel,
        out_type=jax.ShapeDtypeStruct((NCORE, n_pad, H), jnp.float32),
        mesh=mesh,
        scratch_types=[
            pltpu.VMEM((2, NB, C), jnp.int32),
            pltpu.VMEM((2, NB, C), jnp.int32),
            pltpu.VMEM((2, C, H), jnp.float32),
            pltpu.VMEM_SHARED((n_pad, H), jnp.float32),
            pltpu.SemaphoreType.DMA,
            pltpu.SemaphoreType.DMA,
            pltpu.SemaphoreType.DMA,
            pltpu.SemaphoreType.DMA,
        ],
    )
    def seg_sum(feat, src4, dst4, zeros, dummy, out, src_v, dst_v, rows_v,
                acc, gsem0, gsem1, ssem0, ssem1):
        c = lax.axis_index("c")
        s = lax.axis_index("s")
        wid = s * NCORE + c
        base_r = s * span
        gsem = (gsem0, gsem1)
        ssem = (ssem0, ssem1)
        pltpu.sync_copy(zeros.at[pl.ds(base_r, span)], acc.at[pl.ds(base_r, span)])
        plsc.subcore_barrier()

        def group(bank, b, do_drain):
            p = b % 2
            if do_drain:
                pltpu.make_async_copy(rows_v.at[p], acc.at[dst_v.at[bank, b]],
                                      ssem[p]).wait()
            pltpu.async_copy(feat.at[src_v.at[bank, b]], rows_v.at[p],
                             gsem[p]).wait()
            pltpu.async_copy(rows_v.at[p], acc.at[dst_v.at[bank, b]],
                             ssem[p], add=True)

        def load_idx(m, bank):
            pltpu.sync_copy(src4.at[wid, m], src_v.at[bank])
            pltpu.sync_copy(dst4.at[wid, m], dst_v.at[bank])

        load_idx(0, 0)
        for b in range(NB):
            group(0, b, b >= 2)

        def body(m, carry):
            bank = lax.rem(m, 2)
            load_idx(m, bank)
            for b in range(NB):
                group(bank, b, True)
            return carry

        lax.fori_loop(1, m_blocks, body, 0)
        for p in range(2):
            pltpu.make_async_copy(rows_v.at[p], acc.at[dst_v.at[0, p]],
                                  ssem[p]).wait()
        plsc.subcore_barrier()
        pltpu.sync_copy(acc.at[pl.ds(base_r, span)],
                        out.at[c, pl.ds(base_r, span)])

    return seg_sum


def _make_seg_count(n_nodes, n_edges):
    e_per = n_edges // NW
    m_blocks = e_per // (NB * C)
    n_pad = _pad_nodes(n_nodes)
    span = n_pad // NSUB
    mesh = plsc.VectorSubcoreMesh(
        core_axis_name="c", subcore_axis_name="s",
        num_cores=NCORE, num_subcores=NSUB)

    @functools.partial(
        pl.kernel,
        out_type=jax.ShapeDtypeStruct((NCORE, n_pad, H), jnp.float32),
        mesh=mesh,
        scratch_types=[
            pltpu.VMEM((2, NB, C), jnp.int32),
            pltpu.VMEM((C, H), jnp.float32),
            pltpu.VMEM_SHARED((n_pad, H), jnp.float32),
            pltpu.SemaphoreType.DMA,
        ],
    )
    def seg_count(dst4, zeros, ones, out, dst_v, ones_v, acc, ssem):
        c = lax.axis_index("c")
        s = lax.axis_index("s")
        wid = s * NCORE + c
        base_r = s * span
        pltpu.sync_copy(ones, ones_v)
        pltpu.sync_copy(zeros.at[pl.ds(base_r, span)], acc.at[pl.ds(base_r, span)])
        plsc.subcore_barrier()

        def block(m, bank):
            pltpu.sync_copy(dst4.at[wid, m], dst_v.at[bank])
            for b in range(NB):
                pltpu.async_copy(ones_v, acc.at[dst_v.at[bank, b]], ssem,
                                 add=True)
            for b in range(NB):
                pltpu.make_async_copy(ones_v, acc.at[dst_v.at[bank, b]],
                                      ssem).wait()

        block(0, 0)

        def body(m, carry):
            block(m, lax.rem(m, 2))
            return carry

        lax.fori_loop(1, m_blocks, body, 0)
        plsc.subcore_barrier()
        pltpu.sync_copy(acc.at[pl.ds(base_r, span)],
                        out.at[c, pl.ds(base_r, span)])

    return seg_count


def _ln_relu(h, g, b):
    m = jnp.mean(h, axis=-1, keepdims=True)
    v = jnp.mean((h - m) * (h - m), axis=-1, keepdims=True)
    return jnp.maximum((h - m) / jnp.sqrt(v + 1e-5) * g + b, 0.0)


def _dot(a, b):
    return jnp.dot(a, b, preferred_element_type=jnp.float32)


def _make_subgraph(n_poly, n_pts, f_in, rows):
    grid = (n_poly // rows,)

    def body(x_ref, w1, b1, g1, d1, w2, b2, g2, d2, w3, b3, g3, d3, o_ref):
        x = x_ref[...]
        last = x[:, n_pts - 1:n_pts, :]
        if f_in == 2:
            x = x - last
        else:
            ch = lax.broadcasted_iota(jnp.int32, (1, 1, f_in), 2)
            x = x - jnp.where(ch < 2, last, 0.0)
        x = x.reshape(rows * n_pts, f_in)
        h = _ln_relu(_dot(x, w1[...]) + b1[...], g1[...], d1[...])
        h = _ln_relu(_dot(h, w2[...]) + b2[...], g2[...], d2[...])
        h = jnp.max(h.reshape(rows, n_pts, H), axis=1)
        o_ref[...] = _ln_relu(_dot(h, w3[...]) + b3[...], g3[...], d3[...])

    full2 = lambda shape: pl.BlockSpec(shape, lambda i: (0, 0))
    return pl.pallas_call(
        body,
        grid=grid,
        in_specs=[
            pl.BlockSpec((rows, n_pts, f_in), lambda i: (i, 0, 0)),
            full2((f_in, H)), full2((1, H)), full2((1, H)), full2((1, H)),
            full2((H, H)), full2((1, H)), full2((1, H)), full2((1, H)),
            full2((H, H)), full2((1, H)), full2((1, H)), full2((1, H)),
        ],
        out_specs=pl.BlockSpec((rows, H), lambda i: (i, 0)),
        out_shape=jax.ShapeDtypeStruct((n_poly, H), jnp.float32),
    )


def _make_gnn_dense(n_nodes, rows):
    grid = (n_nodes // rows,)

    def body(node_ref, s0, s1, c0, c1, w1n, w1a, b1, g1, d1, w2, b2, g2, d2,
             o_ref):
        node = node_ref[...]
        cnt = c0[0][:, :1] + c1[0][:, :1]
        aggr = (s0[0] + s1[0]) / jnp.maximum(cnt, 1.0)
        h = _dot(node, w1n[...]) + _dot(aggr, w1a[...]) + b1[...]
        h = _ln_relu(h, g1[...], d1[...])
        h = _ln_relu(_dot(h, w2[...]) + b2[...], g2[...], d2[...])
        o_ref[...] = node + h

    full2 = lambda shape: pl.BlockSpec(shape, lambda i: (0, 0))
    return pl.pallas_call(
        body,
        grid=grid,
        in_specs=[
            pl.BlockSpec((rows, H), lambda i: (i, 0)),
            pl.BlockSpec((1, rows, H), lambda i: (0, i, 0)),
            pl.BlockSpec((1, rows, H), lambda i: (1, i, 0)),
            pl.BlockSpec((1, rows, H), lambda i: (0, i, 0)),
            pl.BlockSpec((1, rows, H), lambda i: (1, i, 0)),
            full2((H, H)), full2((H, H)), full2((1, H)), full2((1, H)),
            full2((1, H)),
            full2((H, H)), full2((1, H)), full2((1, H)), full2((1, H)),
        ],
        out_specs=pl.BlockSpec((rows, H), lambda i: (i, 0)),
        out_shape=jax.ShapeDtypeStruct((n_nodes, H), jnp.float32),
    )


def _gather_rows(feat, idx):
    def body(idx_ref, feat_ref, o_ref):
        o_ref[...] = feat_ref[...]

    grid_spec = pltpu.PrefetchScalarGridSpec(
        num_scalar_prefetch=1,
        grid=(B,),
        in_specs=[pl.BlockSpec((1, 1, H),
                               lambda i, idx_ref: (idx_ref[i], 0, 0))],
        out_specs=pl.BlockSpec((1, 1, H), lambda i, idx_ref: (i, 0, 0)),
    )
    out = pl.pallas_call(
        body,
        grid_spec=grid_spec,
        out_shape=jax.ShapeDtypeStruct((B, 1, H), jnp.float32),
    )(idx, feat.reshape(feat.shape[0], 1, H))
    return out.reshape(B, H)


def _heads(tf, tlp_tiled, wt1, bt1, wt2, bt2, wp1, bp1, wp2, bp2):
    TD = K * FS * 2

    def body(tf_ref, tlp_ref, wt1r, bt1r, wt2r, bt2r, wp1r, bp1r, wp2r, bp2r,
             traj_ref, log_ref):
        t = tf_ref[...]
        ht = jnp.maximum(_dot(t, wt1r[...]) + bt1r[...], 0.0)
        traj_ref[...] = _dot(ht, wt2r[...]) + bt2r[...] + tlp_ref[...]
        hp = jnp.maximum(_dot(t, wp1r[...]) + bp1r[...], 0.0)
        log_ref[...] = _dot(hp, wp2r[...]) + bp2r[...]

    return pl.pallas_call(
        body,
        out_shape=(jax.ShapeDtypeStruct((B, TD), jnp.float32),
                   jax.ShapeDtypeStruct((B, K), jnp.float32)),
    )(tf, tlp_tiled, wt1, bt1, wt2, bt2, wp1, bp1, wp2, bp2)


def _enc_args(p):
    out = []
    for l, n in (("l1", "n1"), ("l2", "n2"), ("l3", "n3")):
        out += [p[l]["W"].T, p[l]["b"][None, :], p[n]["g"][None, :],
                p[n]["b"][None, :]]
    return out


def _gnn_args(p):
    w1t = p["l1"]["W"].T
    return [w1t[:H], w1t[H:], p["l1"]["b"][None, :], p["n1"]["g"][None, :],
            p["n1"]["b"][None, :], p["l2"]["W"].T, p["l2"]["b"][None, :],
            p["n2"]["g"][None, :], p["n2"]["b"][None, :]]


def kernel(lane_points, agent_history, edge_index_lane_to_lane,
           edge_index_agent_to_agent, edge_index_lane_to_agent,
           target_agent_global_idx, target_last_pos, params):
    e_ll, e_aa, e_la = (edge_index_lane_to_lane, edge_index_agent_to_agent,
                        edge_index_lane_to_agent)

    sub_lane = _make_subgraph(N_LANE, 20, 2, 200)
    sub_agent = _make_subgraph(N_AGENT, 20, 7, 200)
    lane_feat = sub_lane(lane_points, *_enc_args(params["lane_enc"]))
    agent_feat = sub_agent(agent_history, *_enc_args(params["agent_enc"]))

    zl = jnp.zeros((_pad_nodes(N_LANE), H), jnp.float32)
    za = jnp.zeros((_pad_nodes(N_AGENT), H), jnp.float32)

    def e4(x):
        return x.reshape(NW, -1, NB, C)

    ll_s, ll_d = e4(e_ll[0]), e4(e_ll[1])
    aa_s, aa_d = e4(e_aa[0]), e4(e_aa[1])
    la_s, la_d = e4(e_la[0]), e4(e_la[1])

    ones_c = jnp.ones((C, H), jnp.float32)
    cnt_ll = _make_seg_count(N_LANE, e_ll.shape[1])(ll_d, zl, ones_c)
    cnt_aa = _make_seg_count(N_AGENT, e_aa.shape[1])(aa_d, za, ones_c)
    cnt_la = _make_seg_count(N_AGENT, e_la.shape[1])(la_d, za, ones_c)

    sum_ll = _make_seg_sum(N_LANE, e_ll.shape[1])
    sum_aa = _make_seg_sum(N_AGENT, e_aa.shape[1])
    sum_la = _make_seg_sum(N_AGENT, e_la.shape[1])
    dense_l = _make_gnn_dense(N_LANE, 400)
    dense_a = _make_gnn_dense(N_AGENT, 400)

    for lp in params["layers"]:
        s = sum_ll(lane_feat, ll_s, ll_d, zl, ones_c)
        lane_feat = dense_l(lane_feat, s, s, cnt_ll, cnt_ll,
                            *_gnn_args(lp["ll"]))
        s = sum_aa(agent_feat, aa_s, aa_d, za, ones_c)
        agent_feat = dense_a(agent_feat, s, s, cnt_aa, cnt_aa,
                             *_gnn_args(lp["aa"]))
        s = sum_la(lane_feat, la_s, la_d, za, ones_c)
        agent_feat = dense_a(agent_feat, s, s, cnt_la, cnt_la,
                             *_gnn_args(lp["la"]))

    tf = _gather_rows(agent_feat, target_agent_global_idx)
    tlp_tiled = jnp.tile(target_last_pos, (1, K * FS))
    traj_flat, logits = _heads(
        tf, tlp_tiled,
        params["traj1"]["W"].T, params["traj1"]["b"][None, :],
        params["traj2"]["W"].T, params["traj2"]["b"][None, :],
        params["prob1"]["W"].T, params["prob1"]["b"][None, :],
        params["prob2"]["W"].T, params["prob2"]["b"][None, :])
    pred = traj_flat.reshape(B, K, FS, 2)
    return pred, logits


# speedup vs baseline: 8.3724x; 1.1166x over previous
import functools

import jax
import jax.numpy as jnp
from jax import lax
from jax.experimental import pallas as pl
from jax.experimental.pallas import tpu as pltpu
from jax.experimental.pallas import tpu_sc as plsc

H = 128
K = 6
FS = 30
N_LANE = 10000
N_AGENT = 2000
B = 64
NCORE = 2
NSUB = 16
NW = NCORE * NSUB
C = 125


def _pad_nodes(n):
    return ((n + NSUB * 8 - 1) // (NSUB * 8)) * (NSUB * 8)


NB = 8


def _make_seg_sum(n_nodes, n_edges):
    e_per = n_edges // NW
    m_blocks = e_per // (NB * C)
    n_pad = _pad_nodes(n_nodes)
    span = n_pad // NSUB
    mesh = plsc.VectorSubcoreMesh(
        core_axis_name="c", subcore_axis_name="s",
        num_cores=NCORE, num_subcores=NSUB)

    @functools.partial(
        pl.kernel,
        out_type=jax.ShapeDtypeStruct((NCORE, n_pad, H), jnp.float32),
        mesh=mesh,
        scratch_types=[
            pltpu.VMEM((2, NB, C), jnp.int32),
            pltpu.VMEM((2, NB, C), jnp.int32),
            pltpu.VMEM((2, C, H), jnp.float32),
            pltpu.VMEM_SHARED((n_pad, H), jnp.float32),
            pltpu.SemaphoreType.DMA,
            pltpu.SemaphoreType.DMA,
            pltpu.SemaphoreType.DMA,
            pltpu.SemaphoreType.DMA,
        ],
    )
    def seg_sum(feat, src4, dst4, zeros, dummy, out, src_v, dst_v, rows_v,
                acc, gsem0, gsem1, ssem0, ssem1):
        c = lax.axis_index("c")
        s = lax.axis_index("s")
        wid = s * NCORE + c
        base_r = s * span
        gsem = (gsem0, gsem1)
        ssem = (ssem0, ssem1)
        pltpu.sync_copy(zeros.at[pl.ds(base_r, span)], acc.at[pl.ds(base_r, span)])
        plsc.subcore_barrier()

        def group(bank, b, do_drain):
            p = b % 2
            if do_drain:
                pltpu.make_async_copy(rows_v.at[p], acc.at[dst_v.at[bank, b]],
                                      ssem[p]).wait()
            pltpu.async_copy(feat.at[src_v.at[bank, b]], rows_v.at[p],
                             gsem[p]).wait()
            pltpu.async_copy(rows_v.at[p], acc.at[dst_v.at[bank, b]],
                             ssem[p], add=True)

        def load_idx(m, bank):
            pltpu.sync_copy(src4.at[wid, m], src_v.at[bank])
            pltpu.sync_copy(dst4.at[wid, m], dst_v.at[bank])

        load_idx(0, 0)
        for b in range(NB):
            group(0, b, b >= 2)

        def body(m, carry):
            bank = lax.rem(m, 2)
            load_idx(m, bank)
            for b in range(NB):
                group(bank, b, True)
            return carry

        lax.fori_loop(1, m_blocks, body, 0)
        for p in range(2):
            pltpu.make_async_copy(rows_v.at[p], acc.at[dst_v.at[0, p]],
                                  ssem[p]).wait()
        plsc.subcore_barrier()
        pltpu.sync_copy(acc.at[pl.ds(base_r, span)],
                        out.at[c, pl.ds(base_r, span)])

    return seg_sum


def _make_counts_all(e_ll, e_aa, e_la):
    lp = e_ll // NW
    ap = e_aa // NW
    lap = e_la // NW
    LR = _pad_nodes(N_LANE) // H
    AR = _pad_nodes(N_AGENT) // H
    mesh = plsc.VectorSubcoreMesh(
        core_axis_name="c", subcore_axis_name="s",
        num_cores=NCORE, num_subcores=NSUB)

    @functools.partial(
        pl.kernel,
        out_type=(jax.ShapeDtypeStruct((NCORE, LR, H), jnp.float32),
                  jax.ShapeDtypeStruct((NCORE, AR, H), jnp.float32),
                  jax.ShapeDtypeStruct((NCORE, AR, H), jnp.float32)),
        mesh=mesh,
        compiler_params=pltpu.CompilerParams(needs_layout_passes=False),
        scratch_types=[
            pltpu.VMEM((lp,), jnp.int32),
            pltpu.VMEM((ap,), jnp.int32),
            pltpu.VMEM((lap,), jnp.int32),
            pltpu.VMEM((LR, H), jnp.float32),
            pltpu.VMEM((AR, H), jnp.float32),
            pltpu.VMEM((AR, H), jnp.float32),
            pltpu.VMEM((LR,), jnp.int32),
            pltpu.VMEM((AR,), jnp.int32),
            pltpu.VMEM_SHARED((LR, H), jnp.float32),
            pltpu.VMEM_SHARED((AR, H), jnp.float32),
            pltpu.VMEM_SHARED((AR, H), jnp.float32),
        ],
    )
    def counts(dl2, da2, dla2, zl, za, idl, ida, out_l, out_a, out_la,
               dl_v, da_v, dla_v, hl, ha, hla, il_v, ia_v,
               acc_l, acc_a, acc_la):
        c = lax.axis_index("c")
        s = lax.axis_index("s")
        wid = s * NCORE + c
        pltpu.sync_copy(dl2.at[wid], dl_v)
        pltpu.sync_copy(da2.at[wid], da_v)
        pltpu.sync_copy(dla2.at[wid], dla_v)
        pltpu.sync_copy(zl, hl)
        pltpu.sync_copy(za, ha)
        pltpu.sync_copy(za, hla)
        pltpu.sync_copy(idl, il_v)
        pltpu.sync_copy(ida, ia_v)

        @pl.when(s == 0)
        def _():
            pltpu.sync_copy(zl, acc_l)
            pltpu.sync_copy(za, acc_a)
            pltpu.sync_copy(za, acc_la)

        plsc.subcore_barrier()

        one16 = jnp.ones((16,), jnp.float32)

        def accum(hist, idx_v, n):
            def body(i, carry):
                idx = idx_v[pl.ds(i * 16, 16)]
                row = lax.shift_right_logical(idx, 7)
                col = lax.bitwise_and(idx, 127)
                plsc.addupdate_scatter(hist, [row, col], one16)
                return carry
            lax.fori_loop(0, n // 16, body, 0)

        accum(hl, dl_v, lp)
        accum(ha, da_v, ap)
        accum(hla, dla_v, lap)

        pltpu.sync_copy(hl, acc_l.at[il_v], add=True)
        pltpu.sync_copy(ha, acc_a.at[ia_v], add=True)
        pltpu.sync_copy(hla, acc_la.at[ia_v], add=True)
        plsc.subcore_barrier()

        @pl.when(s == 0)
        def _():
            pltpu.sync_copy(acc_l, out_l.at[c])
            pltpu.sync_copy(acc_a, out_a.at[c])
            pltpu.sync_copy(acc_la, out_la.at[c])

    return counts


def _make_seg_count(n_nodes, n_edges):
    e_per = n_edges // NW
    m_blocks = e_per // (NB * C)
    n_pad = _pad_nodes(n_nodes)
    span = n_pad // NSUB
    mesh = plsc.VectorSubcoreMesh(
        core_axis_name="c", subcore_axis_name="s",
        num_cores=NCORE, num_subcores=NSUB)

    @functools.partial(
        pl.kernel,
        out_type=jax.ShapeDtypeStruct((NCORE, n_pad, H), jnp.float32),
        mesh=mesh,
        scratch_types=[
            pltpu.VMEM((2, NB, C), jnp.int32),
            pltpu.VMEM((C, H), jnp.float32),
            pltpu.VMEM_SHARED((n_pad, H), jnp.float32),
            pltpu.SemaphoreType.DMA,
        ],
    )
    def seg_count(dst4, zeros, ones, out, dst_v, ones_v, acc, ssem):
        c = lax.axis_index("c")
        s = lax.axis_index("s")
        wid = s * NCORE + c
        base_r = s * span
        pltpu.sync_copy(ones, ones_v)
        pltpu.sync_copy(zeros.at[pl.ds(base_r, span)], acc.at[pl.ds(base_r, span)])
        plsc.subcore_barrier()

        def block(m, bank):
            pltpu.sync_copy(dst4.at[wid, m], dst_v.at[bank])
            for b in range(NB):
                pltpu.async_copy(ones_v, acc.at[dst_v.at[bank, b]], ssem,
                                 add=True)
            for b in range(NB):
                pltpu.make_async_copy(ones_v, acc.at[dst_v.at[bank, b]],
                                      ssem).wait()

        block(0, 0)

        def body(m, carry):
            block(m, lax.rem(m, 2))
            return carry

        lax.fori_loop(1, m_blocks, body, 0)
        plsc.subcore_barrier()
        pltpu.sync_copy(acc.at[pl.ds(base_r, span)],
                        out.at[c, pl.ds(base_r, span)])

    return seg_count


def _ln_relu(h, g, b):
    m = jnp.mean(h, axis=-1, keepdims=True)
    v = jnp.mean((h - m) * (h - m), axis=-1, keepdims=True)
    return jnp.maximum((h - m) / jnp.sqrt(v + 1e-5) * g + b, 0.0)


def _dot(a, b):
    return jnp.dot(a, b, preferred_element_type=jnp.float32)


def _make_subgraph(n_poly, n_pts, f_in, rows):
    grid = (n_poly // rows,)

    def body(x_ref, w1, b1, g1, d1, w2, b2, g2, d2, w3, b3, g3, d3, o_ref):
        x = x_ref[...]
        last = x[:, n_pts - 1:n_pts, :]
        if f_in == 2:
            x = x - last
        else:
            ch = lax.broadcasted_iota(jnp.int32, (1, 1, f_in), 2)
            x = x - jnp.where(ch < 2, last, 0.0)
        x = x.reshape(rows * n_pts, f_in)
        h = _ln_relu(_dot(x, w1[...]) + b1[...], g1[...], d1[...])
        h = _ln_relu(_dot(h, w2[...]) + b2[...], g2[...], d2[...])
        h = jnp.max(h.reshape(rows, n_pts, H), axis=1)
        o_ref[...] = _ln_relu(_dot(h, w3[...]) + b3[...], g3[...], d3[...])

    full2 = lambda shape: pl.BlockSpec(shape, lambda i: (0, 0))
    return pl.pallas_call(
        body,
        grid=grid,
        in_specs=[
            pl.BlockSpec((rows, n_pts, f_in), lambda i: (i, 0, 0)),
            full2((f_in, H)), full2((1, H)), full2((1, H)), full2((1, H)),
            full2((H, H)), full2((1, H)), full2((1, H)), full2((1, H)),
            full2((H, H)), full2((1, H)), full2((1, H)), full2((1, H)),
        ],
        out_specs=pl.BlockSpec((rows, H), lambda i: (i, 0)),
        out_shape=jax.ShapeDtypeStruct((n_poly, H), jnp.float32),
    )


def _make_gnn_dense(n_nodes, rows):
    grid = (n_nodes // rows,)

    def body(node_ref, s0, s1, cnt_ref, w1n, w1a, b1, g1, d1, w2, b2, g2,
             d2, o_ref):
        node = node_ref[...]
        cnt = cnt_ref[...][:, :1]
        aggr = (s0[0] + s1[0]) / jnp.maximum(cnt, 1.0)
        h = _dot(node, w1n[...]) + _dot(aggr, w1a[...]) + b1[...]
        h = _ln_relu(h, g1[...], d1[...])
        h = _ln_relu(_dot(h, w2[...]) + b2[...], g2[...], d2[...])
        o_ref[...] = node + h

    full2 = lambda shape: pl.BlockSpec(shape, lambda i: (0, 0))
    return pl.pallas_call(
        body,
        grid=grid,
        in_specs=[
            pl.BlockSpec((rows, H), lambda i: (i, 0)),
            pl.BlockSpec((1, rows, H), lambda i: (0, i, 0)),
            pl.BlockSpec((1, rows, H), lambda i: (1, i, 0)),
            pl.BlockSpec((rows, 8), lambda i: (i, 0)),
            full2((H, H)), full2((H, H)), full2((1, H)), full2((1, H)),
            full2((1, H)),
            full2((H, H)), full2((1, H)), full2((1, H)), full2((1, H)),
        ],
        out_specs=pl.BlockSpec((rows, H), lambda i: (i, 0)),
        out_shape=jax.ShapeDtypeStruct((n_nodes, H), jnp.float32),
    )


def _gather_rows(feat, idx):
    def body(idx_ref, feat_ref, o_ref):
        o_ref[...] = feat_ref[...]

    grid_spec = pltpu.PrefetchScalarGridSpec(
        num_scalar_prefetch=1,
        grid=(B,),
        in_specs=[pl.BlockSpec((1, 1, H),
                               lambda i, idx_ref: (idx_ref[i], 0, 0))],
        out_specs=pl.BlockSpec((1, 1, H), lambda i, idx_ref: (i, 0, 0)),
    )
    out = pl.pallas_call(
        body,
        grid_spec=grid_spec,
        out_shape=jax.ShapeDtypeStruct((B, 1, H), jnp.float32),
    )(idx, feat.reshape(feat.shape[0], 1, H))
    return out.reshape(B, H)


def _heads(tf, tlp_tiled, wt1, bt1, wt2, bt2, wp1, bp1, wp2, bp2):
    TD = K * FS * 2

    def body(tf_ref, tlp_ref, wt1r, bt1r, wt2r, bt2r, wp1r, bp1r, wp2r, bp2r,
             traj_ref, log_ref):
        t = tf_ref[...]
        ht = jnp.maximum(_dot(t, wt1r[...]) + bt1r[...], 0.0)
        traj_ref[...] = _dot(ht, wt2r[...]) + bt2r[...] + tlp_ref[...]
        hp = jnp.maximum(_dot(t, wp1r[...]) + bp1r[...], 0.0)
        log_ref[...] = _dot(hp, wp2r[...]) + bp2r[...]

    return pl.pallas_call(
        body,
        out_shape=(jax.ShapeDtypeStruct((B, TD), jnp.float32),
                   jax.ShapeDtypeStruct((B, K), jnp.float32)),
    )(tf, tlp_tiled, wt1, bt1, wt2, bt2, wp1, bp1, wp2, bp2)


def _enc_args(p):
    out = []
    for l, n in (("l1", "n1"), ("l2", "n2"), ("l3", "n3")):
        out += [p[l]["W"].T, p[l]["b"][None, :], p[n]["g"][None, :],
                p[n]["b"][None, :]]
    return out


def _gnn_args(p):
    w1t = p["l1"]["W"].T
    return [w1t[:H], w1t[H:], p["l1"]["b"][None, :], p["n1"]["g"][None, :],
            p["n1"]["b"][None, :], p["l2"]["W"].T, p["l2"]["b"][None, :],
            p["n2"]["g"][None, :], p["n2"]["b"][None, :]]


def kernel(lane_points, agent_history, edge_index_lane_to_lane,
           edge_index_agent_to_agent, edge_index_lane_to_agent,
           target_agent_global_idx, target_last_pos, params):
    e_ll, e_aa, e_la = (edge_index_lane_to_lane, edge_index_agent_to_agent,
                        edge_index_lane_to_agent)

    sub_lane = _make_subgraph(N_LANE, 20, 2, 200)
    sub_agent = _make_subgraph(N_AGENT, 20, 7, 200)
    lane_feat = sub_lane(lane_points, *_enc_args(params["lane_enc"]))
    agent_feat = sub_agent(agent_history, *_enc_args(params["agent_enc"]))

    zl = jnp.zeros((_pad_nodes(N_LANE), H), jnp.float32)
    za = jnp.zeros((_pad_nodes(N_AGENT), H), jnp.float32)

    def e4(x):
        return x.reshape(NW, -1, NB, C)

    ll_s, ll_d = e4(e_ll[0]), e4(e_ll[1])
    aa_s, aa_d = e4(e_aa[0]), e4(e_aa[1])
    la_s, la_d = e4(e_la[0]), e4(e_la[1])

    ones_c = jnp.ones((C, H), jnp.float32)
    LR = _pad_nodes(N_LANE) // H
    AR = _pad_nodes(N_AGENT) // H
    cl, ca, cla = _make_counts_all(e_ll.shape[1], e_aa.shape[1],
                                   e_la.shape[1])(
        e_ll[1].reshape(NW, -1), e_aa[1].reshape(NW, -1),
        e_la[1].reshape(NW, -1), jnp.zeros((LR, H), jnp.float32),
        jnp.zeros((AR, H), jnp.float32), jnp.arange(LR, dtype=jnp.int32),
        jnp.arange(AR, dtype=jnp.int32))

    def cnt8(c, n):
        tot = c.sum(0).reshape(-1)[:n]
        return jnp.broadcast_to(tot[:, None], (n, 8))

    cnt_ll = cnt8(cl, N_LANE)
    cnt_aa = cnt8(ca, N_AGENT)
    cnt_la = cnt8(cla, N_AGENT)

    sum_ll = _make_seg_sum(N_LANE, e_ll.shape[1])
    sum_aa = _make_seg_sum(N_AGENT, e_aa.shape[1])
    sum_la = _make_seg_sum(N_AGENT, e_la.shape[1])
    dense_l = _make_gnn_dense(N_LANE, 400)
    dense_a = _make_gnn_dense(N_AGENT, 400)

    for lp in params["layers"]:
        s = sum_ll(lane_feat, ll_s, ll_d, zl, ones_c)
        lane_feat = dense_l(lane_feat, s, s, cnt_ll,
                            *_gnn_args(lp["ll"]))
        s = sum_aa(agent_feat, aa_s, aa_d, za, ones_c)
        agent_feat = dense_a(agent_feat, s, s, cnt_aa,
                             *_gnn_args(lp["aa"]))
        s = sum_la(lane_feat, la_s, la_d, za, ones_c)
        agent_feat = dense_a(agent_feat, s, s, cnt_la,
                             *_gnn_args(lp["la"]))

    tf = _gather_rows(agent_feat, target_agent_global_idx)
    tlp_tiled = jnp.tile(target_last_pos, (1, K * FS))
    traj_flat, logits = _heads(
        tf, tlp_tiled,
        params["traj1"]["W"].T, params["traj1"]["b"][None, :],
        params["traj2"]["W"].T, params["traj2"]["b"][None, :],
        params["prob1"]["W"].T, params["prob1"]["b"][None, :],
        params["prob2"]["W"].T, params["prob2"]["b"][None, :])
    pred = traj_flat.reshape(B, K, FS, 2)
    return pred, logits
